# baseline (speedup 1.0000x reference)
import functools

import jax
import jax.numpy as jnp
from jax import lax
from jax.experimental import pallas as pl
from jax.experimental.pallas import tpu as pltpu
from jax.experimental.pallas import tpu_sc as plsc

N_NODES = 10000
N_EDGES = 320000
FEAT = 128
EMB = 64
NCLS = 41
CPAD = 48
VROW = 64
NEG = -1e30

NC, NS = 2, 16
NW = NC * NS

GPW = N_EDGES // NW
GCH = 80
GNCH = GPW // GCH

SPW = N_EDGES // NW
SCH = 80
SNCH = SPW // SCH

ACC_ROWS = 10240
STRIPE = ACC_ROWS // NS


def _embed_tc(features, W_embed, b_embed2d):
    def body(f, w, b, o):
        o[...] = jnp.dot(f[...], w[...], preferred_element_type=jnp.float32) + b[...]

    return pl.pallas_call(
        body,
        out_shape=jax.ShapeDtypeStruct((N_NODES, EMB), jnp.float32),
    )(features, W_embed, b_embed2d)


def _gather_sc(idx4, table):
    mesh = plsc.VectorSubcoreMesh(core_axis_name="c", subcore_axis_name="s")

    @functools.partial(
        pl.kernel,
        out_type=jax.ShapeDtypeStruct((N_EDGES, 2 * EMB), jnp.float32),
        mesh=mesh,
        compiler_params=pltpu.CompilerParams(use_tc_tiling_on_sc=False),
        scratch_types=[
            pltpu.VMEM((2, GNCH, GCH), jnp.int32),
            pltpu.VMEM((2, 2, GCH, EMB), jnp.float32),
            pltpu.SemaphoreType.DMA,
            pltpu.SemaphoreType.DMA,
            pltpu.SemaphoreType.DMA,
            pltpu.SemaphoreType.DMA,
        ],
    )
    def k(idx_hbm, table_hbm, out_hbm, idx_v, rows_v, ss0, ss1, sd0, sd1):
        cid = lax.axis_index("c")
        sid = lax.axis_index("s")
        wid = sid * NC + cid
        base = wid * GPW
        sems = ((ss0, ss1), (sd0, sd1))
        pltpu.sync_copy(idx_hbm.at[:, wid], idx_v)

        def start(kind, j, b):
            pltpu.make_async_copy(
                table_hbm.at[idx_v.at[kind, j]], rows_v.at[kind, b],
                sems[kind][b],
            ).start()

        def finish(kind, j, b):
            pltpu.make_async_copy(
                table_hbm.at[idx_v.at[kind, j]], rows_v.at[kind, b],
                sems[kind][b],
            ).wait()
            pltpu.sync_copy(
                rows_v.at[kind, b],
                out_hbm.at[pl.ds(base + j * GCH, GCH), pl.ds(kind * EMB, EMB)],
            )

        for b in range(2):
            for kind in range(2):
                start(kind, b, b)

        def step(i, carry):
            j0 = 2 * i
            for b in range(2):
                j = j0 + b
                for kind in range(2):
                    finish(kind, j, b)

                    @pl.when(j + 2 < GNCH)
                    def _():
                        start(kind, j + 2, b)

            return carry

        lax.fori_loop(0, (GNCH - 1) // 2, step, 0)
        for kind in range(2):
            finish(kind, GNCH - 1, 0)

    return k(idx4, table)


def _edge_mlp_tc(sd2, W48, b48, wts):
    BE = 6400
    grid = N_EDGES // BE

    def body(sd_ref, w_ref, bt_ref, wt_ref, colt_ref, eye_ref, poss_ref, val_ref):
        sd = sd_ref[...]
        s = sd[:, :EMB]
        d = sd[:, EMB:]
        e1 = (s + d) * 0.5
        dd = s - d
        ecat = jnp.concatenate([e1, dd * dd], axis=1)
        logits_t = lax.dot_general(
            w_ref[...], ecat, (((0,), (1,)), ((), ())),
            preferred_element_type=jnp.float32,
        ) + bt_ref[...]
        mt = jnp.max(logits_t, axis=0, keepdims=True)
        et = jnp.exp(logits_t - mt)
        pt = et / jnp.sum(et, axis=0, keepdims=True)
        poss_ref[...] = pt[:NCLS, :]
        valw_t = (pt + colt_ref[...]) * wt_ref[0]
        left = lax.dot_general(
            valw_t[:, :BE // 2], eye_ref[...], (((0,), (0,)), ((), ())),
            preferred_element_type=jnp.float32,
        )
        right = lax.dot_general(
            valw_t[:, BE // 2:], eye_ref[...], (((0,), (0,)), ((), ())),
            preferred_element_type=jnp.float32,
        )
        val_ref[...] = jnp.concatenate([left, right], axis=1)

    return pl.pallas_call(
        body,
        grid=(grid,),
        in_specs=[
            pl.BlockSpec((BE, 2 * EMB), lambda i: (i, 0)),
            pl.BlockSpec((2 * EMB, CPAD), lambda i: (0, 0)),
            pl.BlockSpec((CPAD, 1), lambda i: (0, 0)),
            pl.BlockSpec((1, 1, BE), lambda i: (i, 0, 0)),
            pl.BlockSpec((CPAD, 1), lambda i: (0, 0)),
            pl.BlockSpec((CPAD, VROW), lambda i: (0, 0)),
        ],
        out_specs=[
            pl.BlockSpec((NCLS, BE), lambda i: (0, i)),
            pl.BlockSpec((BE // 2, 2 * VROW), lambda i: (i, 0)),
        ],
        out_shape=[
            jax.ShapeDtypeStruct((NCLS, N_EDGES), jnp.float32),
            jax.ShapeDtypeStruct((N_EDGES // 2, 2 * VROW), jnp.float32),
        ],
        compiler_params=pltpu.CompilerParams(fuse_transposed_lhs_in_matmul=True),
    )(sd2, W48, b48.reshape(CPAD, 1), wts.reshape(grid, 1, BE),
      (jnp.arange(CPAD) == NCLS).astype(jnp.float32).reshape(CPAD, 1),
      (jnp.arange(CPAD)[:, None] == jnp.arange(VROW)[None, :]).astype(jnp.float32))


def _scatter_sc(value, src2d):
    mesh = plsc.VectorSubcoreMesh(core_axis_name="c", subcore_axis_name="s")

    @functools.partial(
        pl.kernel,
        out_type=jax.ShapeDtypeStruct((NC, ACC_ROWS, VROW), jnp.float32),
        mesh=mesh,
        compiler_params=pltpu.CompilerParams(use_tc_tiling_on_sc=False),
        scratch_types=[
            pltpu.VMEM((SNCH, SCH), jnp.int32),
            pltpu.VMEM((2, SCH, VROW), jnp.float32),
            pltpu.VMEM((128, VROW), jnp.float32),
            pltpu.VMEM_SHARED((ACC_ROWS, VROW), jnp.float32),
            pltpu.SemaphoreType.DMA,
            pltpu.SemaphoreType.DMA,
        ],
    )
    def k(val_hbm, src_hbm, out_hbm, src_v, rows_v, zbuf, acc, sem0, sem1):
        cid = lax.axis_index("c")
        sid = lax.axis_index("s")
        wid = sid * NC + cid
        sems = (sem0, sem1)

        def zrow(r, carry):
            for c in range(VROW // 16):
                zbuf[r, pl.ds(c * 16, 16)] = jnp.zeros((16,), jnp.float32)
            return carry

        lax.fori_loop(0, 128, zrow, 0)
        for t in range(STRIPE // 128):
            pltpu.sync_copy(zbuf, acc.at[pl.ds(sid * STRIPE + t * 128, 128)])
        plsc.subcore_barrier()

        pltpu.sync_copy(src_hbm.at[wid], src_v)
        ebase = wid * SPW
        for b in range(2):
            pltpu.make_async_copy(
                val_hbm.at[pl.ds(ebase + b * SCH, SCH)], rows_v.at[b], sems[b]
            ).start()

        def step(i, carry):
            j0 = 2 * i
            for b in range(2):
                j = j0 + b
                pltpu.make_async_copy(
                    val_hbm.at[pl.ds(ebase + j * SCH, SCH)], rows_v.at[b], sems[b]
                ).wait()
                pltpu.sync_copy(rows_v.at[b], acc.at[src_v.at[j]], add=True)

                @pl.when(j + 2 < SNCH)
                def _():
                    pltpu.make_async_copy(
                        val_hbm.at[pl.ds(ebase + (j + 2) * SCH, SCH)],
                        rows_v.at[b],
                        sems[b],
                    ).start()

            return carry

        lax.fori_loop(0, (SNCH - 1) // 2, step, 0)
        jt = SNCH - 1
        pltpu.make_async_copy(
            val_hbm.at[pl.ds(ebase + jt * SCH, SCH)], rows_v.at[0], sems[0]
        ).wait()
        pltpu.sync_copy(rows_v.at[0], acc.at[src_v.at[jt]], add=True)

        plsc.subcore_barrier()
        pltpu.sync_copy(
            acc.at[pl.ds(sid * STRIPE, STRIPE)],
            out_hbm.at[cid, pl.ds(sid * STRIPE, STRIPE)],
        )

    return k(value, src2d)


def _finalize_tc(acc):
    def body(a_ref, norm_ref, poss_ref):
        a = a_ref[0] + a_ref[1]
        p = a[:N_NODES, :NCLS]
        deg = jnp.maximum(a[:N_NODES, NCLS:NCLS + 1], 1e-12)
        poss_ref[...] = p
        norm_ref[...] = p / deg

    return pl.pallas_call(
        body,
        out_shape=[
            jax.ShapeDtypeStruct((N_NODES, NCLS), jnp.float32),
            jax.ShapeDtypeStruct((N_NODES, NCLS), jnp.float32),
        ],
    )(acc)


def kernel(features, edges, weights, W_embed, b_embed, W_trans, b_trans):
    edges = edges.astype(jnp.int32)
    embed = _embed_tc(features, W_embed, b_embed.reshape(1, EMB))
    idx4 = edges.T.reshape(2, NW, GNCH, GCH)
    sd2 = _gather_sc(idx4, embed)
    W48 = jnp.concatenate(
        [W_trans, jnp.zeros((2 * EMB, CPAD - NCLS), jnp.float32)], axis=1
    )
    b48 = jnp.concatenate(
        [b_trans, jnp.full((CPAD - NCLS,), NEG, jnp.float32)], axis=0
    ).reshape(1, CPAD)
    poss_t, value2 = _edge_mlp_tc(sd2, W48, b48, weights)
    poss_edge = poss_t.T
    BE = 6400
    src_int = (
        edges[:, 0]
        .reshape(N_EDGES // BE, 2, BE // 2)
        .transpose(0, 2, 1)
        .reshape(NW, SNCH, SCH)
    )
    acc = _scatter_sc(value2.reshape(N_EDGES, VROW), src_int)
    norm, poss_node = _finalize_tc(acc)
    return (norm, poss_edge, poss_node)


# speedup vs baseline: 1.2903x; 1.2903x over previous
import functools

import jax
import jax.numpy as jnp
from jax import lax
from jax.experimental import pallas as pl
from jax.experimental.pallas import tpu as pltpu
from jax.experimental.pallas import tpu_sc as plsc

N_NODES = 10000
N_EDGES = 320000
FEAT = 128
EMB = 64
NCLS = 41
CPAD = 48
VROW = 64
NEG = -1e30

NC, NS = 2, 16
NW = NC * NS

NPH = 2
EPH = N_EDGES // NPH

GPW = EPH // NW
GCH = 40
GNCH = GPW // GCH

BE = 6400
BGRID = EPH // BE

SRPW = (EPH // 2) // NW
SCH = 50
SNCH = SRPW // SCH

ACC_ROWS = 10240
STRIPE = ACC_ROWS // NS


def _embed_tc(features, W_embed, b_embed2d):
    def body(f, w, b, o):
        o[...] = jnp.dot(f[...], w[...], preferred_element_type=jnp.float32) + b[...]

    return pl.pallas_call(
        body,
        out_shape=jax.ShapeDtypeStruct((N_NODES, EMB), jnp.float32),
    )(features, W_embed, b_embed2d)


def _gather_sc(idx4, table):
    mesh = plsc.VectorSubcoreMesh(core_axis_name="c", subcore_axis_name="s")

    @functools.partial(
        pl.kernel,
        out_type=jax.ShapeDtypeStruct((EPH, 2 * EMB), jnp.float32),
        mesh=mesh,
        compiler_params=pltpu.CompilerParams(use_tc_tiling_on_sc=False),
        scratch_types=[
            pltpu.VMEM((2, GNCH, GCH), jnp.int32),
            pltpu.VMEM((2, 2, GCH, EMB), jnp.float32),
            pltpu.SemaphoreType.DMA,
            pltpu.SemaphoreType.DMA,
            pltpu.SemaphoreType.DMA,
            pltpu.SemaphoreType.DMA,
        ],
    )
    def k(idx_hbm, table_hbm, out_hbm, idx_v, rows_v, ss0, ss1, sd0, sd1):
        cid = lax.axis_index("c")
        sid = lax.axis_index("s")
        wid = sid * NC + cid
        base = wid * GPW
        sems = ((ss0, ss1), (sd0, sd1))
        pltpu.sync_copy(idx_hbm.at[:, wid], idx_v)

        def start(kind, j, b):
            pltpu.make_async_copy(
                table_hbm.at[idx_v.at[kind, j]], rows_v.at[kind, b],
                sems[kind][b],
            ).start()

        def finish(kind, j, b):
            pltpu.make_async_copy(
                table_hbm.at[idx_v.at[kind, j]], rows_v.at[kind, b],
                sems[kind][b],
            ).wait()
            pltpu.sync_copy(
                rows_v.at[kind, b],
                out_hbm.at[pl.ds(base + j * GCH, GCH), pl.ds(kind * EMB, EMB)],
            )

        for b in range(2):
            for kind in range(2):
                start(kind, b, b)

        def step(i, carry):
            j0 = 2 * i
            for b in range(2):
                j = j0 + b
                for kind in range(2):
                    finish(kind, j, b)

                    @pl.when(j + 2 < GNCH)
                    def _():
                        start(kind, j + 2, b)

            return carry

        lax.fori_loop(0, (GNCH - 1) // 2, step, 0)
        for kind in range(2):
            finish(kind, GNCH - 1, 0)

    return k(idx4, table)


def _edge_mlp_tc(sd2, W48, b48, wts, h, poss_prev):
    def body(*refs):
        sd_ref, w_ref, bt_ref, wt_ref, colt_ref, eye_ref = refs[:6]
        poss_ref, val_ref = refs[-2:]
        sd = sd_ref[...]
        s = sd[:, :EMB]
        d = sd[:, EMB:]
        e1 = (s + d) * 0.5
        dd = s - d
        ecat = jnp.concatenate([e1, dd * dd], axis=1)
        logits_t = lax.dot_general(
            w_ref[...], ecat, (((0,), (1,)), ((), ())),
            preferred_element_type=jnp.float32,
        ) + bt_ref[...]
        mt = jnp.max(logits_t, axis=0, keepdims=True)
        et = jnp.exp(logits_t - mt)
        pt = et / jnp.sum(et, axis=0, keepdims=True)
        poss_ref[...] = pt[:NCLS, :]
        valw_t = (pt + colt_ref[...]) * wt_ref[0]
        left = lax.dot_general(
            valw_t[:, :BE // 2], eye_ref[...], (((0,), (0,)), ((), ())),
            preferred_element_type=jnp.float32,
        )
        right = lax.dot_general(
            valw_t[:, BE // 2:], eye_ref[...], (((0,), (0,)), ((), ())),
            preferred_element_type=jnp.float32,
        )
        val_ref[...] = jnp.concatenate([left, right], axis=1)

    in_specs = [
        pl.BlockSpec((BE, 2 * EMB), lambda i: (i, 0)),
        pl.BlockSpec((2 * EMB, CPAD), lambda i: (0, 0)),
        pl.BlockSpec((CPAD, 1), lambda i: (0, 0)),
        pl.BlockSpec((1, 1, BE), lambda i: (i, 0, 0)),
        pl.BlockSpec((CPAD, 1), lambda i: (0, 0)),
        pl.BlockSpec((CPAD, VROW), lambda i: (0, 0)),
    ]
    ins = [sd2, W48, b48.reshape(CPAD, 1), wts.reshape(BGRID, 1, BE),
           (jnp.arange(CPAD) == NCLS).astype(jnp.float32).reshape(CPAD, 1),
           (jnp.arange(CPAD)[:, None] == jnp.arange(VROW)[None, :]).astype(jnp.float32)]
    aliases = {}
    if poss_prev is not None:
        ins.append(poss_prev)
        in_specs.append(pl.BlockSpec(memory_space=pl.ANY))
        aliases = {6: 0}
    return pl.pallas_call(
        body,
        grid=(BGRID,),
        in_specs=in_specs,
        out_specs=[
            pl.BlockSpec((NCLS, BE), lambda i, h=h: (0, i + h * BGRID)),
            pl.BlockSpec((BE // 2, 2 * VROW), lambda i: (i, 0)),
        ],
        out_shape=[
            jax.ShapeDtypeStruct((NCLS, N_EDGES), jnp.float32),
            jax.ShapeDtypeStruct((EPH // 2, 2 * VROW), jnp.float32),
        ],
        input_output_aliases=aliases,
        compiler_params=pltpu.CompilerParams(fuse_transposed_lhs_in_matmul=True),
    )(*ins)


def _scatter_sc(value, src2d):
    mesh = plsc.VectorSubcoreMesh(core_axis_name="c", subcore_axis_name="s")

    @functools.partial(
        pl.kernel,
        out_type=jax.ShapeDtypeStruct((NC, ACC_ROWS, VROW), jnp.float32),
        mesh=mesh,
        compiler_params=pltpu.CompilerParams(use_tc_tiling_on_sc=False),
        scratch_types=[
            pltpu.VMEM((2, SNCH, SCH), jnp.int32),
            pltpu.VMEM((2, 2, SCH, VROW), jnp.float32),
            pltpu.VMEM((128, VROW), jnp.float32),
            pltpu.VMEM_SHARED((ACC_ROWS, VROW), jnp.float32),
            pltpu.SemaphoreType.DMA,
            pltpu.SemaphoreType.DMA,
            pltpu.SemaphoreType.DMA,
            pltpu.SemaphoreType.DMA,
        ],
    )
    def k(val_hbm, src_hbm, out_hbm, src_v, rows_v, zbuf, acc, sa0, sa1, sb0, sb1):
        cid = lax.axis_index("c")
        sid = lax.axis_index("s")
        wid = sid * NC + cid
        rbase = wid * SRPW
        sems = ((sa0, sa1), (sb0, sb1))

        def zrow(r, carry):
            for c in range(VROW // 16):
                zbuf[r, pl.ds(c * 16, 16)] = jnp.zeros((16,), jnp.float32)
            return carry

        lax.fori_loop(0, 128, zrow, 0)
        for t in range(STRIPE // 128):
            pltpu.sync_copy(zbuf, acc.at[pl.ds(sid * STRIPE + t * 128, 128)])
        plsc.subcore_barrier()

        pltpu.sync_copy(src_hbm.at[:, wid], src_v)

        def start(j, b):
            for h in range(2):
                pltpu.make_async_copy(
                    val_hbm.at[pl.ds(rbase + j * SCH, SCH),
                               pl.ds(h * VROW, VROW)],
                    rows_v.at[h, b], sems[h][b],
                ).start()

        def finish(j, b):
            for h in range(2):
                pltpu.make_async_copy(
                    val_hbm.at[pl.ds(rbase + j * SCH, SCH),
                               pl.ds(h * VROW, VROW)],
                    rows_v.at[h, b], sems[h][b],
                ).wait()
                pltpu.sync_copy(rows_v.at[h, b], acc.at[src_v.at[h, j]], add=True)

        for b in range(2):
            start(b, b)

        def step(i, carry):
            j0 = 2 * i
            for b in range(2):
                j = j0 + b
                finish(j, b)

                @pl.when(j + 2 < SNCH)
                def _():
                    start(j + 2, b)

            return carry

        lax.fori_loop(0, SNCH // 2, step, 0)

        plsc.subcore_barrier()
        pltpu.sync_copy(
            acc.at[pl.ds(sid * STRIPE, STRIPE)],
            out_hbm.at[cid, pl.ds(sid * STRIPE, STRIPE)],
        )

    return k(value, src2d)


def _finalize_tc(acc0, acc1):
    def body(a0_ref, a1_ref, norm_ref, poss_ref):
        a = (a0_ref[0] + a0_ref[1]) + (a1_ref[0] + a1_ref[1])
        p = a[:N_NODES, :NCLS]
        deg = jnp.maximum(a[:N_NODES, NCLS:NCLS + 1], 1e-12)
        poss_ref[...] = p
        norm_ref[...] = p / deg

    return pl.pallas_call(
        body,
        out_shape=[
            jax.ShapeDtypeStruct((N_NODES, NCLS), jnp.float32),
            jax.ShapeDtypeStruct((N_NODES, NCLS), jnp.float32),
        ],
    )(acc0, acc1)


def kernel(features, edges, weights, W_embed, b_embed, W_trans, b_trans):
    edges = edges.astype(jnp.int32)
    embed = _embed_tc(features, W_embed, b_embed.reshape(1, EMB))
    W48 = jnp.concatenate(
        [W_trans, jnp.zeros((2 * EMB, CPAD - NCLS), jnp.float32)], axis=1
    )
    b48 = jnp.concatenate(
        [b_trans, jnp.full((CPAD - NCLS,), NEG, jnp.float32)], axis=0
    ).reshape(1, CPAD)
    e_t = edges.T
    accs, poss_buf = [], None
    for h in range(NPH):
        eh = e_t[:, h * EPH:(h + 1) * EPH]
        sd2 = _gather_sc(eh.reshape(2, NW, GNCH, GCH), embed)
        wts = weights[h * EPH:(h + 1) * EPH]
        poss_buf, value2 = _edge_mlp_tc(sd2, W48, b48, wts, h, poss_buf)
        srcr = eh[0].reshape(BGRID, BE)
        src2 = jnp.concatenate(
            [srcr[:, :BE // 2].reshape(-1), srcr[:, BE // 2:].reshape(-1)]
        ).reshape(2, NW, SNCH, SCH)
        accs.append(_scatter_sc(value2, src2))
    poss_edge = poss_buf.T
    norm, poss_node = _finalize_tc(accs[0], accs[1])
    return (norm, poss_edge, poss_node)
